# Initial kernel scaffold; baseline (speedup 1.0000x reference)
import jax
import jax.numpy as jnp
from jax.experimental import pallas as pl


def kernel(query_val, source_val, source_state):
    raise NotImplementedError("write your pallas kernel here")


# speedup vs baseline: 31.4477x; 31.4477x over previous
import functools

import jax
import jax.numpy as jnp
from jax.experimental import pallas as pl
from jax.experimental.pallas import tpu as pltpu

TOPK_K = 64
QB = 256

INT_MIN = -(2**31)
INT_MAX = 2**31 - 1


def _body(q_ref, svT_ref, val_ref, state_ref, dv_ref, ds_ref):
    scores = jnp.dot(q_ref[0], svT_ref[0], preferred_element_type=jnp.float32)

    bits = jax.lax.bitcast_convert_type(scores, jnp.int32)
    keys = jnp.where(bits >= 0, bits, bits ^ jnp.int32(INT_MAX))

    lo0 = jnp.full((QB, 1), INT_MIN, dtype=jnp.int32)
    hi0 = jnp.full((QB, 1), INT_MAX, dtype=jnp.int32)

    def step(_, carry):
        lo, hi = carry
        mid = (lo >> 1) + (hi >> 1) + (lo & hi & 1) + ((lo ^ hi) & 1)
        cnt = jnp.sum((keys >= mid).astype(jnp.int32), axis=1, keepdims=True)
        pred = cnt >= TOPK_K
        return jnp.where(pred, mid, lo), jnp.where(pred, hi, mid - 1)

    lo, hi = jax.lax.fori_loop(0, 32, step, (lo0, hi0))

    mask = keys >= lo
    edges = jnp.where(mask, scores / (1.0 + jnp.abs(scores)), 0.0)

    edges_bf = edges.astype(jnp.bfloat16)
    dv_ref[0] = jnp.dot(edges_bf, val_ref[0], preferred_element_type=jnp.float32)

    ds = jnp.sum(edges * state_ref[0], axis=1, keepdims=True)
    ds_ref[0, 0] = jnp.broadcast_to(ds, (QB, 8))


@jax.jit
def kernel(query_val, source_val, source_state):
    B, Nq, D = query_val.shape
    Ns = source_val.shape[1]
    nqb = Nq // QB

    q_bf = query_val.astype(jnp.bfloat16)
    sv_bf = source_val.astype(jnp.bfloat16)
    svT_bf = sv_bf.swapaxes(1, 2)
    state3 = source_state[:, None, :]

    grid = (B, nqb)
    dv, ds = pl.pallas_call(
        _body,
        grid=grid,
        in_specs=[
            pl.BlockSpec((1, QB, D), lambda b, i: (b, i, 0)),
            pl.BlockSpec((1, D, Ns), lambda b, i: (b, 0, 0)),
            pl.BlockSpec((1, Ns, D), lambda b, i: (b, 0, 0)),
            pl.BlockSpec((1, 1, Ns), lambda b, i: (b, 0, 0)),
        ],
        out_specs=[
            pl.BlockSpec((1, QB, D), lambda b, i: (b, i, 0)),
            pl.BlockSpec((1, 1, QB, 8), lambda b, i: (b, i, 0, 0)),
        ],
        out_shape=[
            jax.ShapeDtypeStruct((B, Nq, D), jnp.float32),
            jax.ShapeDtypeStruct((B, nqb, QB, 8), jnp.float32),
        ],
    )(q_bf, svT_bf, sv_bf, state3)

    delta_state = ds[..., 0].reshape(B, Nq)
    return (delta_state, dv)
